# baseline (speedup 1.0000x reference)
import jax
import jax.numpy as jnp
from jax.experimental import pallas as pl


def _concat_body(r_ref, g_ref, o_ref):
    n = r_ref.shape[1]
    o_ref[:, :n, :] = r_ref[...]
    o_ref[:, n:, :] = g_ref[...]


def kernel(rois, gt_boxes):
    B, N, C = rois.shape
    _, G, _ = gt_boxes.shape
    return pl.pallas_call(
        _concat_body,
        out_shape=jax.ShapeDtypeStruct((B, N + G, C), rois.dtype),
    )(rois, gt_boxes)
